# Initial kernel scaffold; baseline (speedup 1.0000x reference)
import jax
import jax.numpy as jnp
from jax.experimental import pallas as pl


def kernel(z, edge_index):
    raise NotImplementedError("write your pallas kernel here")


# speedup vs baseline: 2.9432x; 2.9432x over previous
import functools

import jax
import jax.numpy as jnp
from jax import lax
from jax.experimental import pallas as pl
from jax.experimental.pallas import tpu as pltpu
from jax.experimental.pallas import tpu_sc as plsc

E = 160000
D = 256
NLANE = 16
NW = 32
CHUNK = 128
EPW = 5120
E_PAD = NW * EPW
NCHUNK_FULL = EPW // CHUNK
LAST_COUNT = E - 31 * EPW
NCHUNK_LAST = LAST_COUNT // CHUNK

_mesh = plsc.VectorSubcoreMesh(core_axis_name="c", subcore_axis_name="s")


@functools.partial(
    pl.kernel,
    out_type=jax.ShapeDtypeStruct((E,), jnp.float32),
    mesh=_mesh,
    compiler_params=pltpu.CompilerParams(needs_layout_passes=False),
    scratch_types=[
        pltpu.VMEM((EPW,), jnp.int32),
        pltpu.VMEM((EPW,), jnp.int32),
        pltpu.VMEM((CHUNK, D), jnp.float32),
        pltpu.VMEM((CHUNK, D), jnp.float32),
        pltpu.VMEM((EPW,), jnp.float32),
        pltpu.VMEM((NLANE * NLANE,), jnp.float32),
        pltpu.SemaphoreType.DMA,
        pltpu.SemaphoreType.DMA,
    ],
)
def _decode(z_hbm, src_hbm, dst_hbm, out_hbm,
            sidx, didx, srows, drows, obuf, ttile, sem_s, sem_d):
    wid = lax.axis_index("s") * 2 + lax.axis_index("c")
    base = wid * EPW
    nchunk = jnp.where(wid == NW - 1, NCHUNK_LAST, NCHUNK_FULL)

    pltpu.sync_copy(src_hbm.at[pl.ds(base, EPW)], sidx)
    pltpu.sync_copy(dst_hbm.at[pl.ds(base, EPW)], didx)

    lane = lax.iota(jnp.int32, NLANE)

    def chunk_body(c, carry):
        off = c * CHUNK
        cp_s = pltpu.async_copy(z_hbm.at[sidx.at[pl.ds(off, CHUNK)]], srows, sem_s)
        cp_d = pltpu.async_copy(z_hbm.at[didx.at[pl.ds(off, CHUNK)]], drows, sem_d)
        cp_s.wait()
        cp_d.wait()

        def group_body(g, carry2):
            gbase = g * NLANE
            for e in range(NLANE):
                row = gbase + e
                acc = srows[row, pl.ds(0, NLANE)] * drows[row, pl.ds(0, NLANE)]
                for k in range(1, D // NLANE):
                    acc = acc + (srows[row, pl.ds(k * NLANE, NLANE)]
                                 * drows[row, pl.ds(k * NLANE, NLANE)])
                plsc.store_scatter(ttile, [e * NLANE + ((lane + e) & 15)], acc)
            res = plsc.load_gather(ttile, [lane * NLANE + (lane & 15)])
            for c in range(1, NLANE):
                res = res + plsc.load_gather(
                    ttile, [lane * NLANE + ((lane + c) & 15)])
            obuf[pl.ds(off + gbase, NLANE)] = 1.0 / (1.0 + jnp.exp(-res))
            return carry2

        lax.fori_loop(0, CHUNK // NLANE, group_body, 0)
        return carry

    lax.fori_loop(0, nchunk, chunk_body, 0)

    @pl.when(wid < NW - 1)
    def _():
        pltpu.sync_copy(obuf.at[pl.ds(0, EPW)], out_hbm.at[pl.ds(base, EPW)])

    @pl.when(wid == NW - 1)
    def _():
        pltpu.sync_copy(obuf.at[pl.ds(0, LAST_COUNT)],
                        out_hbm.at[pl.ds((NW - 1) * EPW, LAST_COUNT)])


def kernel(z, edge_index):
    src = jnp.pad(edge_index[0].astype(jnp.int32), (0, E_PAD - E))
    dst = jnp.pad(edge_index[1].astype(jnp.int32), (0, E_PAD - E))
    return _decode(z, src, dst)
